# Initial kernel scaffold; baseline (speedup 1.0000x reference)
import jax
import jax.numpy as jnp
from jax.experimental import pallas as pl


def kernel(x, table):
    raise NotImplementedError("write your pallas kernel here")


# speedup vs baseline: 2.4947x; 2.4947x over previous
import jax
import jax.numpy as jnp
from jax.experimental import pallas as pl
from jax.experimental.pallas import tpu as pltpu
from jax.experimental.pallas import tpu_sc as plsc

_EMBED_DIM = 64
_PAIR_DIM = 2 * _EMBED_DIM
_WINDOW = 128


def _gather_kernel(npairs: int):
    mesh = plsc.VectorSubcoreMesh(core_axis_name="core", subcore_axis_name="subcore")

    @pl.kernel(
        out_type=jax.ShapeDtypeStruct((npairs, _PAIR_DIM), jnp.float32),
        mesh=mesh,
    )
    def kern(table_hbm, idx_hbm, out_hbm):
        def body(i_vmem, o_vmem):
            pltpu.sync_copy(table_hbm.at[i_vmem.at[0]], o_vmem)

        pltpu.emit_pipeline(
            body,
            grid=(npairs // _WINDOW,),
            in_specs=[pl.BlockSpec((1, _WINDOW), index_map=lambda i: (0, i))],
            out_specs=[
                pl.BlockSpec((_WINDOW, _PAIR_DIM), index_map=lambda i: (i, 0))
            ],
            core_axis_name=("core", "subcore"),
            dimension_semantics=(pltpu.PARALLEL,),
        )(idx_hbm, out_hbm)

    return kern


def kernel(x, table):
    orig_shape = x.shape
    n = x.size
    npairs = n // 2
    xi = x.reshape(npairs, 2).astype(jnp.int32)
    fused = (xi[:, 0] * 8 + xi[:, 1]).reshape(1, npairs)
    pair_table = jnp.concatenate(
        [
            jnp.broadcast_to(table[:, None, :], (8, 8, _EMBED_DIM)),
            jnp.broadcast_to(table[None, :, :], (8, 8, _EMBED_DIM)),
        ],
        axis=-1,
    ).reshape(64, _PAIR_DIM)
    out = _gather_kernel(npairs)(pair_table, fused)
    return out.reshape(*orig_shape, _EMBED_DIM)


# speedup vs baseline: 4.5490x; 1.8235x over previous
import jax
import jax.numpy as jnp
from jax.experimental import pallas as pl
from jax.experimental.pallas import tpu as pltpu
from jax.experimental.pallas import tpu_sc as plsc

_EMBED_DIM = 64
_FUSE = 4
_PAIR_DIM = _FUSE * _EMBED_DIM
_WINDOW = 128


def _gather_kernel(npairs: int):
    mesh = plsc.VectorSubcoreMesh(core_axis_name="core", subcore_axis_name="subcore")

    @pl.kernel(
        out_type=jax.ShapeDtypeStruct((npairs, _PAIR_DIM), jnp.float32),
        mesh=mesh,
    )
    def kern(table_hbm, idx_hbm, out_hbm):
        def body(i_vmem, o_vmem):
            pltpu.sync_copy(table_hbm.at[i_vmem.at[0]], o_vmem)

        pltpu.emit_pipeline(
            body,
            grid=(npairs // _WINDOW,),
            in_specs=[pl.BlockSpec((1, _WINDOW), index_map=lambda i: (0, i))],
            out_specs=[
                pl.BlockSpec((_WINDOW, _PAIR_DIM), index_map=lambda i: (i, 0))
            ],
            core_axis_name=("core", "subcore"),
            dimension_semantics=(pltpu.PARALLEL,),
        )(idx_hbm, out_hbm)

    return kern


def kernel(x, table):
    orig_shape = x.shape
    n = x.size
    npairs = n // _FUSE
    xi = x.reshape(npairs, _FUSE).astype(jnp.int32)
    weights = jnp.array(
        [8 ** (_FUSE - 1 - i) for i in range(_FUSE)], dtype=jnp.int32
    )
    fused = (xi * weights).sum(axis=1).reshape(1, npairs)
    nrows = 8**_FUSE
    parts = []
    for i in range(_FUSE):
        shape = [1] * _FUSE + [_EMBED_DIM]
        shape[i] = 8
        parts.append(
            jnp.broadcast_to(
                table.reshape(shape), [8] * _FUSE + [_EMBED_DIM]
            )
        )
    pair_table = jnp.concatenate(parts, axis=-1).reshape(nrows, _PAIR_DIM)
    out = _gather_kernel(npairs)(pair_table, fused)
    return out.reshape(*orig_shape, _EMBED_DIM)


# speedup vs baseline: 4.5714x; 1.0049x over previous
import jax
import jax.numpy as jnp
from jax import lax
from jax.experimental import pallas as pl
from jax.experimental.pallas import tpu as pltpu
from jax.experimental.pallas import tpu_sc as plsc

_EMBED_DIM = 64
_FUSE = 2
_PAIR_DIM = _FUSE * _EMBED_DIM
_WINDOW = 128


def _gather_kernel(npairs: int):
    mesh = plsc.VectorSubcoreMesh(core_axis_name="core", subcore_axis_name="subcore")

    @pl.kernel(
        out_type=jax.ShapeDtypeStruct((npairs, _PAIR_DIM), jnp.float32),
        mesh=mesh,
        scratch_types=[pltpu.VMEM_SHARED((8**_FUSE, _PAIR_DIM), jnp.float32)],
    )
    def kern(table_hbm, idx_hbm, out_hbm, tab_vmem):
        @pl.when(lax.axis_index("subcore") == 0)
        def _():
            pltpu.sync_copy(table_hbm, tab_vmem)

        plsc.subcore_barrier()

        def body(i_vmem, o_vmem):
            pltpu.sync_copy(tab_vmem.at[i_vmem.at[0]], o_vmem)

        pltpu.emit_pipeline(
            body,
            grid=(npairs // _WINDOW,),
            in_specs=[pl.BlockSpec((1, _WINDOW), index_map=lambda i: (0, i))],
            out_specs=[
                pl.BlockSpec((_WINDOW, _PAIR_DIM), index_map=lambda i: (i, 0))
            ],
            core_axis_name=("core", "subcore"),
            dimension_semantics=(pltpu.PARALLEL,),
        )(idx_hbm, out_hbm)

    return kern


def kernel(x, table):
    orig_shape = x.shape
    n = x.size
    npairs = n // _FUSE
    xi = x.reshape(npairs, _FUSE).astype(jnp.int32)
    weights = jnp.array(
        [8 ** (_FUSE - 1 - i) for i in range(_FUSE)], dtype=jnp.int32
    )
    fused = (xi * weights).sum(axis=1).reshape(1, npairs)
    nrows = 8**_FUSE
    parts = []
    for i in range(_FUSE):
        shape = [1] * _FUSE + [_EMBED_DIM]
        shape[i] = 8
        parts.append(
            jnp.broadcast_to(
                table.reshape(shape), [8] * _FUSE + [_EMBED_DIM]
            )
        )
    pair_table = jnp.concatenate(parts, axis=-1).reshape(nrows, _PAIR_DIM)
    out = _gather_kernel(npairs)(pair_table, fused)
    return out.reshape(*orig_shape, _EMBED_DIM)


# speedup vs baseline: 9.1669x; 2.0053x over previous
import dataclasses

import jax
import jax.numpy as jnp
from jax.experimental import pallas as pl
from jax.experimental.pallas import tpu as pltpu
from jax.experimental.pallas import tpu_sc as plsc

_EMBED_DIM = 64
_NVOCAB = 8
_LANES = 16
_CHUNK = 512


def _expand_kernel(nrows: int, ncols: int):
    mesh = plsc.VectorSubcoreMesh(core_axis_name="core", subcore_axis_name="subcore")
    cp = pltpu.CompilerParams()
    if "needs_layout_passes" in pltpu.CompilerParams.__dataclass_fields__:
        cp = dataclasses.replace(cp, needs_layout_passes=False)

    @pl.kernel(
        out_type=jax.ShapeDtypeStruct((ncols, _EMBED_DIM, nrows), jnp.float32),
        mesh=mesh,
        scratch_types=[pltpu.VMEM((_NVOCAB * _EMBED_DIM,), jnp.float32)],
        compiler_params=cp,
    )
    def kern(tabt_hbm, xt_hbm, out_hbm, tab_vmem):
        pltpu.sync_copy(tabt_hbm, tab_vmem)

        def body(x_vmem, o_vmem):
            @pl.loop(0, _CHUNK, step=_LANES)
            def _(g):
                xv = x_vmem[0, pl.ds(g, _LANES)]
                for d in range(_EMBED_DIM):
                    o_vmem[0, d, pl.ds(g, _LANES)] = plsc.load_gather(
                        tab_vmem, [xv + (d * _NVOCAB)]
                    )

        pltpu.emit_pipeline(
            body,
            grid=(ncols, nrows // _CHUNK),
            in_specs=[pl.BlockSpec((1, _CHUNK), index_map=lambda j, b: (j, b))],
            out_specs=[
                pl.BlockSpec(
                    (1, _EMBED_DIM, _CHUNK), index_map=lambda j, b: (j, 0, b)
                )
            ],
            core_axis_name=("core", "subcore"),
            dimension_semantics=(pltpu.PARALLEL, pltpu.PARALLEL),
        )(xt_hbm, out_hbm)

    return kern


def kernel(x, table):
    nrows, ncols = x.shape
    xt = x.astype(jnp.int32).T
    tabt = table.T.reshape(_NVOCAB * _EMBED_DIM)
    out_t = _expand_kernel(nrows, ncols)(tabt, xt)
    return jnp.transpose(out_t, (2, 0, 1))


# speedup vs baseline: 44.8573x; 4.8934x over previous
import dataclasses

import jax
import jax.numpy as jnp
from jax.experimental import pallas as pl
from jax.experimental.pallas import tpu as pltpu
from jax.experimental.pallas import tpu_sc as plsc

_EMBED_DIM = 64
_NVOCAB = 8
_LANES = 16
_CHUNK = 512


def _expand_kernel(nrows: int, ncols: int):
    mesh = plsc.VectorSubcoreMesh(core_axis_name="core", subcore_axis_name="subcore")
    cp = pltpu.CompilerParams()
    if "needs_layout_passes" in pltpu.CompilerParams.__dataclass_fields__:
        cp = dataclasses.replace(cp, needs_layout_passes=False)

    @pl.kernel(
        out_type=jax.ShapeDtypeStruct((ncols, _EMBED_DIM, nrows), jnp.float32),
        mesh=mesh,
        scratch_types=[pltpu.VMEM((_NVOCAB * _EMBED_DIM,), jnp.float32)],
        compiler_params=cp,
    )
    def kern(tabt_hbm, xt_hbm, out_hbm, tab_vmem):
        pltpu.sync_copy(tabt_hbm, tab_vmem)

        def body(x_vmem, o_vmem):
            @plsc.parallel_loop(0, _CHUNK, step=_LANES)
            def _(g):
                xv = x_vmem[0, pl.ds(g, _LANES)]
                for d in range(_EMBED_DIM):
                    o_vmem[0, d, pl.ds(g, _LANES)] = plsc.load_gather(
                        tab_vmem, [xv + (d * _NVOCAB)]
                    )

        pltpu.emit_pipeline(
            body,
            grid=(ncols, nrows // _CHUNK),
            in_specs=[pl.BlockSpec((1, _CHUNK), index_map=lambda j, b: (j, b))],
            out_specs=[
                pl.BlockSpec(
                    (1, _EMBED_DIM, _CHUNK), index_map=lambda j, b: (j, 0, b)
                )
            ],
            core_axis_name=("core", "subcore"),
            dimension_semantics=(pltpu.PARALLEL, pltpu.PARALLEL),
        )(xt_hbm, out_hbm)

    return kern


def kernel(x, table):
    nrows, ncols = x.shape
    xt = x.astype(jnp.int32).T
    tabt = table.T.reshape(_NVOCAB * _EMBED_DIM)
    out_t = _expand_kernel(nrows, ncols)(tabt, xt)
    return jnp.transpose(out_t, (2, 0, 1))
